# baseline (speedup 1.0000x reference)
import functools

import jax
import jax.numpy as jnp
from jax import lax
from jax.experimental import pallas as pl
from jax.experimental.pallas import tpu as pltpu
from jax.experimental.pallas import tpu_sc as plsc

_D = 64
_B = 4096 * 200
_NW = 32
_BW = _B // _NW
_CHUNK = 256
_NCHUNK = _BW // _CHUNK
_SCALE = 8.0

_mesh = plsc.VectorSubcoreMesh(core_axis_name="c", subcore_axis_name="s")


@functools.partial(
    pl.kernel,
    out_type=jax.ShapeDtypeStruct((_B, _D), jnp.float32),
    mesh=_mesh,
    compiler_params=pltpu.CompilerParams(needs_layout_passes=False),
    scratch_types=[
        pltpu.VMEM((_NCHUNK, _CHUNK), jnp.int32),
        pltpu.VMEM((_CHUNK,), jnp.int32),
        pltpu.VMEM((_CHUNK, 2 * _D), jnp.float32),
        pltpu.VMEM((_CHUNK, _D), jnp.float32),
        pltpu.SemaphoreType.DMA,
    ],
)
def _embed(x_hbm, table_hbm, out_hbm, idx_v, half_v, rows_v, out_v, gsem):
    wid = lax.axis_index("s") * 2 + lax.axis_index("c")
    base = wid * _BW
    pltpu.sync_copy(x_hbm.at[wid], idx_v)
    lane = lax.iota(jnp.int32, 16)

    def chunk_body(g, carry):
        def half_body(q, carry2):
            v16 = idx_v[g, pl.ds(q * 16, 16)]
            half_v[pl.ds(q * 16, 16)] = v16 >> 1
            return carry2

        lax.fori_loop(0, _CHUNK // 16, half_body, 0, unroll=4)
        pltpu.async_copy(table_hbm.at[half_v], rows_v, gsem).wait()

        def compact_body(q, carry2):
            tok16 = q * 16 + lane
            v16 = idx_v[g, pl.ds(q * 16, 16)]
            col0 = (v16 & 1) * _D
            for j in range(_D):
                vals = plsc.load_gather(rows_v, [tok16, col0 + j])
                plsc.store_scatter(out_v, [tok16, lane * 0 + j], vals * _SCALE)
            return carry2

        lax.fori_loop(0, _CHUNK // 16, compact_body, 0)
        pltpu.sync_copy(out_v, out_hbm.at[pl.ds(base + g * _CHUNK, _CHUNK)])
        return carry

    lax.fori_loop(0, _NCHUNK, chunk_body, 0)


def kernel(x, table):
    xw = x.reshape(_NW, _NCHUNK, _CHUNK).astype(jnp.int32)
    tpack = table.reshape(500_000, 2 * _D)
    out = _embed(xw, tpack)
    return out.reshape(x.shape[0], x.shape[1], _D)


# speedup vs baseline: 2.3921x; 2.3921x over previous
import functools

import jax
import jax.numpy as jnp
from jax import lax
from jax.experimental import pallas as pl
from jax.experimental.pallas import tpu as pltpu
from jax.experimental.pallas import tpu_sc as plsc

_D = 64
_B = 4096 * 200
_NW = 32
_BW = _B // _NW
_CHUNK = 160
_NCHUNK = _BW // _CHUNK
_NPAIR = _NCHUNK // 2
_SCALE = 8.0

_mesh = plsc.VectorSubcoreMesh(core_axis_name="c", subcore_axis_name="s")


@functools.partial(
    pl.kernel,
    out_type=jax.ShapeDtypeStruct((_B, _D), jnp.float32),
    mesh=_mesh,
    compiler_params=pltpu.CompilerParams(needs_layout_passes=False),
    scratch_types=[
        pltpu.VMEM((_NCHUNK, _CHUNK), jnp.int32),
        pltpu.VMEM((_CHUNK,), jnp.int32),
        pltpu.VMEM((_CHUNK,), jnp.int32),
        pltpu.VMEM((_CHUNK, 2 * _D), jnp.float32),
        pltpu.VMEM((_CHUNK, 2 * _D), jnp.float32),
        pltpu.VMEM((_CHUNK, _D), jnp.float32),
        pltpu.VMEM((_CHUNK, _D), jnp.float32),
        pltpu.SemaphoreType.DMA,
        pltpu.SemaphoreType.DMA,
        pltpu.SemaphoreType.DMA,
        pltpu.SemaphoreType.DMA,
    ],
)
def _embed(x_hbm, table_hbm, out_hbm, idx_v, half_a, half_b, rows_a, rows_b,
           outv_a, outv_b, gsa, gsb, osa, osb):
    wid = lax.axis_index("s") * 2 + lax.axis_index("c")
    base = wid * _BW
    pltpu.sync_copy(x_hbm.at[wid], idx_v)
    lane = lax.iota(jnp.int32, 16)

    def start_gather(half_v, rows_v, g, sem):
        def half_body(q, carry):
            v16 = idx_v[g, pl.ds(q * 16, 16)]
            half_v[pl.ds(q * 16, 16)] = v16 >> 1
            return carry

        lax.fori_loop(0, _CHUNK // 16, half_body, 0, unroll=4)
        pltpu.async_copy(table_hbm.at[half_v], rows_v, sem)

    def wait_gather(half_v, rows_v, sem):
        pltpu.make_async_copy(table_hbm.at[half_v], rows_v, sem).wait()

    def compact(rows_v, out_v, g):
        def q_body(q, carry):
            tok16 = q * 16 + lane
            v16 = idx_v[g, pl.ds(q * 16, 16)]
            col0 = (v16 & 1) * _D
            for j in range(_D):
                cvec = (lane + j) & (_D - 1)
                vals = plsc.load_gather(rows_v, [tok16, col0 + cvec])
                plsc.store_scatter(out_v, [tok16, cvec], vals * _SCALE)
            return carry

        lax.fori_loop(0, _CHUNK // 16, q_body, 0)

    def start_out(out_v, g, sem):
        pltpu.async_copy(
            out_v, out_hbm.at[pl.ds(base + g * _CHUNK, _CHUNK)], sem
        )

    def wait_out(out_v, sem):
        pltpu.make_async_copy(
            out_v, out_hbm.at[pl.ds(base, _CHUNK)], sem
        ).wait()

    start_gather(half_a, rows_a, 0, gsa)

    def pair_body(h, carry):
        e = 2 * h
        o = e + 1
        start_gather(half_b, rows_b, o, gsb)
        wait_gather(half_a, rows_a, gsa)
        pl.when(h > 0)(lambda: wait_out(outv_a, osa))
        compact(rows_a, outv_a, e)
        start_out(outv_a, e, osa)
        pl.when(h < _NPAIR - 1)(
            lambda: start_gather(half_a, rows_a, e + 2, gsa))
        wait_gather(half_b, rows_b, gsb)
        pl.when(h > 0)(lambda: wait_out(outv_b, osb))
        compact(rows_b, outv_b, o)
        start_out(outv_b, o, osb)
        return carry

    lax.fori_loop(0, _NPAIR, pair_body, 0)
    wait_out(outv_a, osa)
    wait_out(outv_b, osb)


def kernel(x, table):
    xw = x.reshape(_NW, _NCHUNK, _CHUNK).astype(jnp.int32)
    tpack = table.reshape(500_000, 2 * _D)
    out = _embed(xw, tpack)
    return out.reshape(x.shape[0], x.shape[1], _D)
